# speedup vs baseline: 2.1833x; 2.1833x over previous
import functools

import jax
import jax.numpy as jnp
from jax import lax
from jax.experimental import pallas as pl
from jax.experimental.pallas import tpu as pltpu
from jax.experimental.pallas import tpu_sc as plsc

NC = 2
NS = 16
NW = NC * NS


def _make_gather(V, D, B, C):
    assert B % NW == 0
    b_per_w = B // NW
    assert b_per_w % C == 0
    chunks = b_per_w // C
    mesh = plsc.VectorSubcoreMesh(core_axis_name="c", subcore_axis_name="s")

    @functools.partial(
        pl.kernel,
        mesh=mesh,
        out_type=jax.ShapeDtypeStruct((B, D), jnp.float32),
        scratch_types=[
            pltpu.VMEM((chunks, C), jnp.int32),
            pltpu.VMEM((C, D), jnp.float32),
            pltpu.SemaphoreType.DMA,
        ],
    )
    def gather_kernel(table_hbm, idx_hbm, out_hbm, idx_v, rows_v, sem):
        wid = lax.axis_index("s") * NC + lax.axis_index("c")
        base = wid * b_per_w
        pltpu.sync_copy(idx_hbm.at[wid], idx_v)

        def body(g, carry):
            pltpu.async_copy(table_hbm.at[idx_v.at[g]], rows_v, sem).wait()
            pltpu.sync_copy(rows_v, out_hbm.at[pl.ds(base + g * C, C)])
            return carry

        lax.fori_loop(0, chunks, body, 0)

    return gather_kernel


def kernel(position_ids, pe_weight):
    V, D = pe_weight.shape
    orig_shape = position_ids.shape
    B = position_ids.size
    C = 64
    idx3 = position_ids.astype(jnp.int32).reshape(NW, (B // NW) // C, C)
    out = _make_gather(V, D, B, C)(pe_weight, idx3)
    return out.reshape(orig_shape + (D,))


# speedup vs baseline: 2.3774x; 1.0889x over previous
import functools

import jax
import jax.numpy as jnp
from jax import lax
from jax.experimental import pallas as pl
from jax.experimental.pallas import tpu as pltpu
from jax.experimental.pallas import tpu_sc as plsc

NC = 2
NS = 16
NW = NC * NS


def _make_gather(V, D, B, C, NBUF):
    assert B % NW == 0
    b_per_w = B // NW
    assert b_per_w % C == 0
    chunks = b_per_w // C
    assert chunks % NBUF == 0 and chunks >= 2 * NBUF
    mesh = plsc.VectorSubcoreMesh(core_axis_name="c", subcore_axis_name="s")

    scratch = [pltpu.VMEM((chunks, C), jnp.int32)]
    scratch += [pltpu.VMEM((C, D), jnp.float32) for _ in range(NBUF)]
    scratch += [pltpu.SemaphoreType.DMA for _ in range(2 * NBUF)]

    @functools.partial(
        pl.kernel,
        mesh=mesh,
        out_type=jax.ShapeDtypeStruct((B, D), jnp.float32),
        scratch_types=scratch,
    )
    def gather_kernel(table_hbm, idx_hbm, out_hbm, idx_v, *bufs_and_sems):
        bufs = bufs_and_sems[:NBUF]
        in_sems = bufs_and_sems[NBUF:2 * NBUF]
        out_sems = bufs_and_sems[2 * NBUF:]
        wid = lax.axis_index("s") * NC + lax.axis_index("c")
        base = wid * b_per_w
        pltpu.sync_copy(idx_hbm.at[wid], idx_v)

        def gather_into(c, b):
            pltpu.async_copy(table_hbm.at[idx_v.at[c]], bufs[b], in_sems[b])

        def out_slice(c):
            return out_hbm.at[pl.ds(base + c * C, C)]

        for b in range(NBUF - 1):
            gather_into(b, b)

        def body(i, carry):
            g = i * NBUF
            for b in range(NBUF):
                c = g + b
                pltpu.make_async_copy(
                    table_hbm.at[idx_v.at[c]], bufs[b], in_sems[b]).wait()
                pltpu.async_copy(bufs[b], out_slice(c), out_sems[b])
                f = c + (NBUF - 1)
                bf = (b + NBUF - 1) % NBUF

                @pl.when(jnp.logical_and(f < chunks, c > 0))
                def _():
                    pltpu.make_async_copy(
                        bufs[bf], out_slice(c - 1), out_sems[bf]).wait()

                @pl.when(f < chunks)
                def _():
                    gather_into(f, bf)
            return carry

        lax.fori_loop(0, chunks // NBUF, body, 0)

        for b in range(NBUF):
            pltpu.make_async_copy(
                bufs[b], out_slice(chunks - NBUF + b), out_sems[b]).wait()

    return gather_kernel


def kernel(position_ids, pe_weight):
    V, D = pe_weight.shape
    orig_shape = position_ids.shape
    B = position_ids.size
    C, NBUF = 16, 4
    idx3 = position_ids.astype(jnp.int32).reshape(NW, (B // NW) // C, C)
    out = _make_gather(V, D, B, C, NBUF)(pe_weight, idx3)
    return out.reshape(orig_shape + (D,))
